# Initial kernel scaffold; baseline (speedup 1.0000x reference)
import jax
import jax.numpy as jnp
from jax.experimental import pallas as pl


def kernel(kg_enc_input, entity_embedding, rel_embedding, W_mlp, b_mlp):
    raise NotImplementedError("write your pallas kernel here")


# speedup vs baseline: 2.1030x; 2.1030x over previous
import functools

import jax
import jax.numpy as jnp
from jax import lax
from jax.experimental import pallas as pl
from jax.experimental.pallas import tpu as pltpu
from jax.experimental.pallas import tpu_sc as plsc

NC = 2
NS = 16
NW = NC * NS
CHUNK = 64
D = 300
DP = 320


def _pad_body(x_ref, o_ref):
    o_ref[...] = jnp.concatenate(
        [x_ref[...], jnp.zeros((x_ref.shape[0], DP - D), jnp.float32)], axis=1)


def _pad_table(tbl, rows_per_blk=1000):
    V = tbl.shape[0]
    return pl.pallas_call(
        _pad_body,
        grid=(V // rows_per_blk,),
        in_specs=[pl.BlockSpec((rows_per_blk, D), lambda i: (i, 0))],
        out_specs=pl.BlockSpec((rows_per_blk, DP), lambda i: (i, 0)),
        out_shape=jax.ShapeDtypeStruct((V, DP), jnp.float32),
    )(tbl)


def _make_gather(B):
    per_w = B // NW
    n_chunks = per_w // CHUNK
    mesh = plsc.VectorSubcoreMesh(core_axis_name="c", subcore_axis_name="s")

    @functools.partial(
        pl.kernel,
        out_type=jax.ShapeDtypeStruct((3, B, DP), jnp.float32),
        mesh=mesh,
        scratch_types=[
            pltpu.VMEM((per_w,), jnp.int32),
            pltpu.VMEM((per_w,), jnp.int32),
            pltpu.VMEM((per_w,), jnp.int32),
            pltpu.VMEM((CHUNK, DP), jnp.float32),
            pltpu.VMEM((CHUNK, DP), jnp.float32),
            pltpu.VMEM((CHUNK, DP), jnp.float32),
            pltpu.SemaphoreType.DMA,
            pltpu.SemaphoreType.DMA,
            pltpu.SemaphoreType.DMA,
        ],
        compiler_params=pltpu.CompilerParams(use_tc_tiling_on_sc=False),
    )
    def gather_k(head_hbm, rel_hbm, tail_hbm, ent_hbm, rtab_hbm, out_hbm,
                 idx_h, idx_r, idx_t, buf0, buf1, buf2, sem0, sem1, sem2):
        wid = lax.axis_index("s") * NC + lax.axis_index("c")
        base = wid * per_w
        pltpu.sync_copy(head_hbm.at[pl.ds(base, per_w)], idx_h)
        pltpu.sync_copy(rel_hbm.at[pl.ds(base, per_w)], idx_r)
        pltpu.sync_copy(tail_hbm.at[pl.ds(base, per_w)], idx_t)

        def body(c, carry):
            off = base + c * CHUNK
            cs = pl.ds(c * CHUNK, CHUNK)
            cp0 = pltpu.async_copy(ent_hbm.at[idx_h.at[cs]], buf0, sem0)
            cp1 = pltpu.async_copy(rtab_hbm.at[idx_r.at[cs]], buf1, sem1)
            cp2 = pltpu.async_copy(ent_hbm.at[idx_t.at[cs]], buf2, sem2)
            cp0.wait()
            pltpu.sync_copy(buf0, out_hbm.at[0, pl.ds(off, CHUNK)])
            cp1.wait()
            pltpu.sync_copy(buf1, out_hbm.at[1, pl.ds(off, CHUNK)])
            cp2.wait()
            pltpu.sync_copy(buf2, out_hbm.at[2, pl.ds(off, CHUNK)])
            return carry

        lax.fori_loop(0, n_chunks, body, 0)

    return gather_k


def _mm_body(x_ref, w_ref, b_ref, o_ref):
    w = w_ref[...]
    acc = lax.dot_general(x_ref[0], w[:, 0:DP],
                          (((1,), (1,)), ((), ())),
                          preferred_element_type=jnp.float32)
    acc += lax.dot_general(x_ref[1], w[:, DP:2 * DP],
                           (((1,), (1,)), ((), ())),
                           preferred_element_type=jnp.float32)
    acc += lax.dot_general(x_ref[2], w[:, 2 * DP:3 * DP],
                           (((1,), (1,)), ((), ())),
                           preferred_element_type=jnp.float32)
    o_ref[...] = acc + b_ref[...]


def _matmul(gathered, W_pad, b_mlp, B, tm=512, interpret=False):
    grid = (B // tm,)
    return pl.pallas_call(
        _mm_body,
        grid=grid,
        in_specs=[
            pl.BlockSpec((3, tm, DP), lambda i: (0, i, 0)),
            pl.BlockSpec((3 * D, 3 * DP), lambda i: (0, 0)),
            pl.BlockSpec((1, 3 * D), lambda i: (0, 0)),
        ],
        out_specs=pl.BlockSpec((tm, 3 * D), lambda i: (i, 0)),
        out_shape=jax.ShapeDtypeStruct((B, 3 * D), jnp.float32),
        interpret=interpret,
    )(gathered, W_pad, b_mlp.reshape(1, 3 * D))


def kernel(kg_enc_input, entity_embedding, rel_embedding, W_mlp, b_mlp):
    batch, n_turns, n_triples, _ = kg_enc_input.shape
    B = batch * n_turns * n_triples
    idx = kg_enc_input.reshape(B, 3)
    head = idx[:, 0]
    rel = idx[:, 1]
    tail = idx[:, 2]
    ent_pad = _pad_table(entity_embedding)
    rtab_pad = _pad_table(rel_embedding)
    W_pad = jnp.pad(W_mlp.reshape(3 * D, 3, D), ((0, 0), (0, 0), (0, DP - D)))
    W_pad = W_pad.reshape(3 * D, 3 * DP)
    gathered = _make_gather(B)(head, rel, tail, ent_pad, rtab_pad)
    out = _matmul(gathered, W_pad, b_mlp, B)
    return out.reshape(batch, n_turns, n_triples, 3 * D)


# speedup vs baseline: 3.0197x; 1.4359x over previous
import functools

import jax
import jax.numpy as jnp
from jax import lax
from jax.experimental import pallas as pl
from jax.experimental.pallas import tpu as pltpu
from jax.experimental.pallas import tpu_sc as plsc

NC = 2
NS = 16
NW = NC * NS
CHUNK = 64
D = 300
DP = 384


def _pad_body(x_ref, o_ref):
    o_ref[...] = jnp.concatenate(
        [x_ref[...], jnp.zeros((x_ref.shape[0], DP - D), jnp.float32)], axis=1)


def _pad_table(tbl, rows_per_blk=1000):
    V = tbl.shape[0]
    return pl.pallas_call(
        _pad_body,
        grid=(V // rows_per_blk,),
        in_specs=[pl.BlockSpec((rows_per_blk, D), lambda i: (i, 0))],
        out_specs=pl.BlockSpec((rows_per_blk, DP), lambda i: (i, 0)),
        out_shape=jax.ShapeDtypeStruct((V, DP), jnp.float32),
    )(tbl)


def _make_gather(B):
    per_w = B // NW
    n_chunks = per_w // CHUNK
    mesh = plsc.VectorSubcoreMesh(core_axis_name="c", subcore_axis_name="s")

    @functools.partial(
        pl.kernel,
        out_type=jax.ShapeDtypeStruct((3, B, DP), jnp.float32),
        mesh=mesh,
        scratch_types=[
            pltpu.VMEM((per_w,), jnp.int32),
            pltpu.VMEM((per_w,), jnp.int32),
            pltpu.VMEM((per_w,), jnp.int32),
            pltpu.VMEM((CHUNK, DP), jnp.float32),
            pltpu.VMEM((CHUNK, DP), jnp.float32),
            pltpu.VMEM((CHUNK, DP), jnp.float32),
            pltpu.SemaphoreType.DMA,
            pltpu.SemaphoreType.DMA,
            pltpu.SemaphoreType.DMA,
        ],
    )
    def gather_k(head_hbm, rel_hbm, tail_hbm, ent_hbm, rtab_hbm, out_hbm,
                 idx_h, idx_r, idx_t, buf0, buf1, buf2, sem0, sem1, sem2):
        wid = lax.axis_index("s") * NC + lax.axis_index("c")
        base = wid * per_w
        pltpu.sync_copy(head_hbm.at[pl.ds(base, per_w)], idx_h)
        pltpu.sync_copy(rel_hbm.at[pl.ds(base, per_w)], idx_r)
        pltpu.sync_copy(tail_hbm.at[pl.ds(base, per_w)], idx_t)

        def body(c, carry):
            off = base + c * CHUNK
            cs = pl.ds(c * CHUNK, CHUNK)
            cp0 = pltpu.async_copy(ent_hbm.at[idx_h.at[cs]], buf0, sem0)
            cp1 = pltpu.async_copy(rtab_hbm.at[idx_r.at[cs]], buf1, sem1)
            cp2 = pltpu.async_copy(ent_hbm.at[idx_t.at[cs]], buf2, sem2)
            cp0.wait()
            pltpu.sync_copy(buf0, out_hbm.at[0, pl.ds(off, CHUNK)])
            cp1.wait()
            pltpu.sync_copy(buf1, out_hbm.at[1, pl.ds(off, CHUNK)])
            cp2.wait()
            pltpu.sync_copy(buf2, out_hbm.at[2, pl.ds(off, CHUNK)])
            return carry

        lax.fori_loop(0, n_chunks, body, 0)

    return gather_k


def _mm_body(x_ref, w_ref, b_ref, o_ref):
    w = w_ref[...]
    acc = lax.dot_general(x_ref[0], w[:, 0:DP],
                          (((1,), (1,)), ((), ())),
                          preferred_element_type=jnp.float32)
    acc += lax.dot_general(x_ref[1], w[:, DP:2 * DP],
                           (((1,), (1,)), ((), ())),
                           preferred_element_type=jnp.float32)
    acc += lax.dot_general(x_ref[2], w[:, 2 * DP:3 * DP],
                           (((1,), (1,)), ((), ())),
                           preferred_element_type=jnp.float32)
    o_ref[...] = acc + b_ref[...]


def _matmul(gathered, W_pad, b_mlp, B, tm=512, interpret=False):
    grid = (B // tm,)
    return pl.pallas_call(
        _mm_body,
        grid=grid,
        in_specs=[
            pl.BlockSpec((3, tm, DP), lambda i: (0, i, 0)),
            pl.BlockSpec((3 * D, 3 * DP), lambda i: (0, 0)),
            pl.BlockSpec((1, 3 * D), lambda i: (0, 0)),
        ],
        out_specs=pl.BlockSpec((tm, 3 * D), lambda i: (i, 0)),
        out_shape=jax.ShapeDtypeStruct((B, 3 * D), jnp.float32),
        interpret=interpret,
    )(gathered, W_pad, b_mlp.reshape(1, 3 * D))


def kernel(kg_enc_input, entity_embedding, rel_embedding, W_mlp, b_mlp):
    batch, n_turns, n_triples, _ = kg_enc_input.shape
    B = batch * n_turns * n_triples
    idx = kg_enc_input.reshape(B, 3)
    head = idx[:, 0]
    rel = idx[:, 1]
    tail = idx[:, 2]
    ent_pad = _pad_table(entity_embedding)
    rtab_pad = _pad_table(rel_embedding)
    W_pad = jnp.pad(W_mlp.reshape(3 * D, 3, D), ((0, 0), (0, 0), (0, DP - D)))
    W_pad = W_pad.reshape(3 * D, 3 * DP)
    gathered = _make_gather(B)(head, rel, tail, ent_pad, rtab_pad)
    out = _matmul(gathered, W_pad, b_mlp, B)
    return out.reshape(batch, n_turns, n_triples, 3 * D)


# speedup vs baseline: 3.9875x; 1.3205x over previous
import functools

import jax
import jax.numpy as jnp
from jax import lax
from jax.experimental import pallas as pl
from jax.experimental.pallas import tpu as pltpu
from jax.experimental.pallas import tpu_sc as plsc

NC = 2
NS = 16
NW = NC * NS
CHUNK = 64
D = 300
DP = 384


def _pad_body(xt_ref, o_ref):
    xt = xt_ref[...]
    o_ref[...] = jnp.concatenate(
        [xt.T, jnp.zeros((xt.shape[1], DP - D), jnp.float32)], axis=1)


def _pad_table(tbl_t, rows_per_blk=2048):
    V = tbl_t.shape[1]
    return pl.pallas_call(
        _pad_body,
        grid=(pl.cdiv(V, rows_per_blk),),
        in_specs=[pl.BlockSpec((D, rows_per_blk), lambda i: (0, i))],
        out_specs=pl.BlockSpec((rows_per_blk, DP), lambda i: (i, 0)),
        out_shape=jax.ShapeDtypeStruct((V, DP), jnp.float32),
    )(tbl_t)


def _make_gather(B):
    per_w = B // NW
    n_chunks = per_w // CHUNK
    mesh = plsc.VectorSubcoreMesh(core_axis_name="c", subcore_axis_name="s")

    @functools.partial(
        pl.kernel,
        out_type=jax.ShapeDtypeStruct((3, B, DP), jnp.float32),
        mesh=mesh,
        scratch_types=[
            pltpu.VMEM((per_w,), jnp.int32),
            pltpu.VMEM((per_w,), jnp.int32),
            pltpu.VMEM((per_w,), jnp.int32),
            pltpu.VMEM((CHUNK, DP), jnp.float32),
            pltpu.VMEM((CHUNK, DP), jnp.float32),
            pltpu.VMEM((CHUNK, DP), jnp.float32),
            pltpu.SemaphoreType.DMA,
            pltpu.SemaphoreType.DMA,
            pltpu.SemaphoreType.DMA,
        ],
    )
    def gather_k(head_hbm, rel_hbm, tail_hbm, ent_hbm, rtab_hbm, out_hbm,
                 idx_h, idx_r, idx_t, buf0, buf1, buf2, sem0, sem1, sem2):
        wid = lax.axis_index("s") * NC + lax.axis_index("c")
        base = wid * per_w
        pltpu.sync_copy(head_hbm.at[pl.ds(base, per_w)], idx_h)
        pltpu.sync_copy(rel_hbm.at[pl.ds(base, per_w)], idx_r)
        pltpu.sync_copy(tail_hbm.at[pl.ds(base, per_w)], idx_t)

        def body(c, carry):
            off = base + c * CHUNK
            cs = pl.ds(c * CHUNK, CHUNK)
            cp0 = pltpu.async_copy(ent_hbm.at[idx_h.at[cs]], buf0, sem0)
            cp1 = pltpu.async_copy(rtab_hbm.at[idx_r.at[cs]], buf1, sem1)
            cp2 = pltpu.async_copy(ent_hbm.at[idx_t.at[cs]], buf2, sem2)
            cp0.wait()
            pltpu.sync_copy(buf0, out_hbm.at[0, pl.ds(off, CHUNK)])
            cp1.wait()
            pltpu.sync_copy(buf1, out_hbm.at[1, pl.ds(off, CHUNK)])
            cp2.wait()
            pltpu.sync_copy(buf2, out_hbm.at[2, pl.ds(off, CHUNK)])
            return carry

        lax.fori_loop(0, n_chunks, body, 0)

    return gather_k


def _mm_body(x_ref, w_ref, b_ref, o_ref):
    w = w_ref[...]
    acc = lax.dot_general(x_ref[0], w[:, 0:DP],
                          (((1,), (1,)), ((), ())),
                          preferred_element_type=jnp.float32)
    acc += lax.dot_general(x_ref[1], w[:, DP:2 * DP],
                           (((1,), (1,)), ((), ())),
                           preferred_element_type=jnp.float32)
    acc += lax.dot_general(x_ref[2], w[:, 2 * DP:3 * DP],
                           (((1,), (1,)), ((), ())),
                           preferred_element_type=jnp.float32)
    o_ref[...] = acc + b_ref[...]


def _matmul(gathered, W_pad, b_mlp, B, tm=512, interpret=False):
    grid = (B // tm,)
    return pl.pallas_call(
        _mm_body,
        grid=grid,
        in_specs=[
            pl.BlockSpec((3, tm, DP), lambda i: (0, i, 0)),
            pl.BlockSpec((3 * D, 3 * DP), lambda i: (0, 0)),
            pl.BlockSpec((1, 3 * D), lambda i: (0, 0)),
        ],
        out_specs=pl.BlockSpec((tm, 3 * D), lambda i: (i, 0)),
        out_shape=jax.ShapeDtypeStruct((B, 3 * D), jnp.float32),
        interpret=interpret,
    )(gathered, W_pad, b_mlp.reshape(1, 3 * D))


def kernel(kg_enc_input, entity_embedding, rel_embedding, W_mlp, b_mlp):
    batch, n_turns, n_triples, _ = kg_enc_input.shape
    B = batch * n_turns * n_triples
    idx = kg_enc_input.reshape(B, 3)
    head = idx[:, 0]
    rel = idx[:, 1]
    tail = idx[:, 2]
    ent_pad = _pad_table(entity_embedding.T)
    rtab_pad = _pad_table(rel_embedding.T)
    W_pad = jnp.pad(W_mlp.reshape(3 * D, 3, D), ((0, 0), (0, 0), (0, DP - D)))
    W_pad = W_pad.reshape(3 * D, 3 * DP)
    gathered = _make_gather(B)(head, rel, tail, ent_pad, rtab_pad)
    out = _matmul(gathered, W_pad, b_mlp, B)
    return out.reshape(batch, n_turns, n_triples, 3 * D)


# speedup vs baseline: 4.2668x; 1.0701x over previous
import functools

import jax
import jax.numpy as jnp
from jax import lax
from jax.experimental import pallas as pl
from jax.experimental.pallas import tpu as pltpu
from jax.experimental.pallas import tpu_sc as plsc

NC = 2
NS = 16
NW = NC * NS
CHUNK = 64
D = 300
DP = 384


def _pad_body(xt_ref, o_ref):
    xt = xt_ref[...]
    o_ref[...] = jnp.concatenate(
        [xt.T, jnp.zeros((xt.shape[1], DP - D), jnp.float32)], axis=1)


def _pad_table(tbl_t, rows_per_blk=2048):
    V = tbl_t.shape[1]
    return pl.pallas_call(
        _pad_body,
        grid=(pl.cdiv(V, rows_per_blk),),
        in_specs=[pl.BlockSpec((D, rows_per_blk), lambda i: (0, i))],
        out_specs=pl.BlockSpec((rows_per_blk, DP), lambda i: (i, 0)),
        out_shape=jax.ShapeDtypeStruct((V, DP), jnp.float32),
    )(tbl_t)


def _make_gather(B):
    per_w = B // NW
    n_chunks = per_w // CHUNK
    mesh = plsc.VectorSubcoreMesh(core_axis_name="c", subcore_axis_name="s")

    @functools.partial(
        pl.kernel,
        out_type=jax.ShapeDtypeStruct((3, B, DP), jnp.float32),
        mesh=mesh,
        scratch_types=[
            pltpu.VMEM((per_w,), jnp.int32),
            pltpu.VMEM((per_w,), jnp.int32),
            pltpu.VMEM((per_w,), jnp.int32),
            pltpu.VMEM((CHUNK, DP), jnp.float32),
            pltpu.VMEM((CHUNK, DP), jnp.float32),
            pltpu.VMEM((CHUNK, DP), jnp.float32),
            pltpu.SemaphoreType.DMA,
            pltpu.SemaphoreType.DMA,
            pltpu.SemaphoreType.DMA,
        ],
    )
    def gather_k(head_hbm, rel_hbm, tail_hbm, ent_hbm, rtab_hbm, out_hbm,
                 idx_h, idx_r, idx_t, buf0, buf1, buf2, sem0, sem1, sem2):
        wid = lax.axis_index("s") * NC + lax.axis_index("c")
        base = wid * per_w
        pltpu.sync_copy(head_hbm.at[pl.ds(base, per_w)], idx_h)
        pltpu.sync_copy(rel_hbm.at[pl.ds(base, per_w)], idx_r)
        pltpu.sync_copy(tail_hbm.at[pl.ds(base, per_w)], idx_t)

        def body(c, carry):
            off = base + c * CHUNK
            cs = pl.ds(c * CHUNK, CHUNK)
            cp0 = pltpu.async_copy(ent_hbm.at[idx_h.at[cs]], buf0, sem0)
            cp1 = pltpu.async_copy(rtab_hbm.at[idx_r.at[cs]], buf1, sem1)
            cp2 = pltpu.async_copy(ent_hbm.at[idx_t.at[cs]], buf2, sem2)
            cp0.wait()
            pltpu.sync_copy(buf0, out_hbm.at[0, pl.ds(off, CHUNK)])
            cp1.wait()
            pltpu.sync_copy(buf1, out_hbm.at[1, pl.ds(off, CHUNK)])
            cp2.wait()
            pltpu.sync_copy(buf2, out_hbm.at[2, pl.ds(off, CHUNK)])
            return carry

        lax.fori_loop(0, n_chunks, body, 0)

    return gather_k


def _mm_body(x_ref, w_ref, b_ref, o_ref):
    w = w_ref[...]
    acc = lax.dot_general(w[:, 0:DP], x_ref[0],
                          (((1,), (1,)), ((), ())),
                          preferred_element_type=jnp.float32)
    acc += lax.dot_general(w[:, DP:2 * DP], x_ref[1],
                           (((1,), (1,)), ((), ())),
                           preferred_element_type=jnp.float32)
    acc += lax.dot_general(w[:, 2 * DP:3 * DP], x_ref[2],
                           (((1,), (1,)), ((), ())),
                           preferred_element_type=jnp.float32)
    acc += b_ref[...]
    o_ref[...] = acc.reshape(1, 3 * D, 8, 128)


def _matmul(gathered, W_pad, b_mlp, n_triples, batch, interpret=False):
    grid = (n_triples,)
    return pl.pallas_call(
        _mm_body,
        grid=grid,
        in_specs=[
            pl.BlockSpec((3, batch, DP), lambda i: (0, i, 0)),
            pl.BlockSpec((3 * D, 3 * DP), lambda i: (0, 0)),
            pl.BlockSpec((3 * D, 1), lambda i: (0, 0)),
        ],
        out_specs=pl.BlockSpec((1, 3 * D, 8, 128), lambda i: (i, 0, 0, 0)),
        out_shape=jax.ShapeDtypeStruct(
            (n_triples, 3 * D, batch // 128, 128), jnp.float32),
        interpret=interpret,
    )(gathered, W_pad, b_mlp.reshape(3 * D, 1))


def kernel(kg_enc_input, entity_embedding, rel_embedding, W_mlp, b_mlp):
    batch, n_turns, n_triples, _ = kg_enc_input.shape
    B = batch * n_turns * n_triples
    idx_t = kg_enc_input.reshape(batch, n_turns * n_triples, 3)
    idx_t = idx_t.transpose(1, 2, 0)
    head = idx_t[:, 0, :].reshape(B)
    rel = idx_t[:, 1, :].reshape(B)
    tail = idx_t[:, 2, :].reshape(B)
    ent_pad = _pad_table(entity_embedding.T)
    rtab_pad = _pad_table(rel_embedding.T)
    W_pad = jnp.pad(W_mlp.reshape(3 * D, 3, D), ((0, 0), (0, 0), (0, DP - D)))
    W_pad = W_pad.reshape(3 * D, 3 * DP)
    gathered = _make_gather(B)(head, rel, tail, ent_pad, rtab_pad)
    out = _matmul(gathered, W_pad, b_mlp, n_turns * n_triples, batch)
    out = out.reshape(n_turns * n_triples, 3 * D, batch)
    out = out.transpose(2, 0, 1).reshape(batch, n_turns, n_triples, 3 * D)
    return out


# speedup vs baseline: 4.2844x; 1.0041x over previous
import functools

import jax
import jax.numpy as jnp
from jax import lax
from jax.experimental import pallas as pl
from jax.experimental.pallas import tpu as pltpu
from jax.experimental.pallas import tpu_sc as plsc

NC = 2
NS = 16
NW = NC * NS
CHUNK = 64
D = 300
DP = 384


def _pad_body(xt_ref, o_ref):
    xt = xt_ref[...]
    o_ref[...] = jnp.concatenate(
        [xt.T, jnp.zeros((xt.shape[1], DP - D), jnp.float32)], axis=1)


def _pad_table(tbl_t, rows_per_blk=2048):
    V = tbl_t.shape[1]
    return pl.pallas_call(
        _pad_body,
        grid=(pl.cdiv(V, rows_per_blk),),
        in_specs=[pl.BlockSpec((D, rows_per_blk), lambda i: (0, i))],
        out_specs=pl.BlockSpec((rows_per_blk, DP), lambda i: (i, 0)),
        out_shape=jax.ShapeDtypeStruct((V, DP), jnp.float32),
    )(tbl_t)


def _make_gather(B, n_parts):
    per_w = B // NW
    n_chunks = per_w // CHUNK
    mesh = plsc.VectorSubcoreMesh(core_axis_name="c", subcore_axis_name="s")

    idx_scratch = [pltpu.VMEM((per_w,), jnp.int32)] * n_parts
    buf_scratch = [pltpu.VMEM((CHUNK, DP), jnp.float32)] * n_parts
    sem_scratch = [pltpu.SemaphoreType.DMA] * n_parts

    @functools.partial(
        pl.kernel,
        out_type=jax.ShapeDtypeStruct((n_parts, B, DP), jnp.float32),
        mesh=mesh,
        scratch_types=idx_scratch + buf_scratch + sem_scratch,
    )
    def gather_k(*args):
        idx_hbm = args[:n_parts]
        tbl_hbm = args[n_parts]
        out_hbm = args[n_parts + 1]
        idx_v = args[n_parts + 2:2 * n_parts + 2]
        bufs = args[2 * n_parts + 2:3 * n_parts + 2]
        sems = args[3 * n_parts + 2:]
        wid = lax.axis_index("s") * NC + lax.axis_index("c")
        base = wid * per_w
        for j in range(n_parts):
            pltpu.sync_copy(idx_hbm[j].at[pl.ds(base, per_w)], idx_v[j])

        def body(c, carry):
            off = base + c * CHUNK
            cs = pl.ds(c * CHUNK, CHUNK)
            cps = [pltpu.async_copy(tbl_hbm.at[idx_v[j].at[cs]], bufs[j],
                                    sems[j])
                   for j in range(n_parts)]
            for j in range(n_parts):
                cps[j].wait()
                pltpu.sync_copy(bufs[j], out_hbm.at[j, pl.ds(off, CHUNK)])
            return carry

        lax.fori_loop(0, n_chunks, body, 0)

    return gather_k


def _mm_body(xet_ref, xr_ref, w_ref, b_ref, o_ref):
    w = w_ref[...]
    acc = lax.dot_general(w[:, 0:DP], xet_ref[0],
                          (((1,), (1,)), ((), ())),
                          preferred_element_type=jnp.float32)
    acc += lax.dot_general(w[:, DP:2 * DP], xr_ref[0],
                           (((1,), (1,)), ((), ())),
                           preferred_element_type=jnp.float32)
    acc += lax.dot_general(w[:, 2 * DP:3 * DP], xet_ref[1],
                           (((1,), (1,)), ((), ())),
                           preferred_element_type=jnp.float32)
    acc += b_ref[...]
    o_ref[...] = acc.reshape(1, 3 * D, 8, 128)


def _matmul(g_et, g_r, W_pad, b_mlp, n_triples, batch, interpret=False):
    grid = (n_triples,)
    return pl.pallas_call(
        _mm_body,
        grid=grid,
        in_specs=[
            pl.BlockSpec((2, batch, DP), lambda i: (0, i, 0)),
            pl.BlockSpec((1, batch, DP), lambda i: (0, i, 0)),
            pl.BlockSpec((3 * D, 3 * DP), lambda i: (0, 0)),
            pl.BlockSpec((3 * D, 1), lambda i: (0, 0)),
        ],
        out_specs=pl.BlockSpec((1, 3 * D, 8, 128), lambda i: (i, 0, 0, 0)),
        out_shape=jax.ShapeDtypeStruct(
            (n_triples, 3 * D, batch // 128, 128), jnp.float32),
        interpret=interpret,
    )(g_et, g_r, W_pad, b_mlp.reshape(3 * D, 1))


def kernel(kg_enc_input, entity_embedding, rel_embedding, W_mlp, b_mlp):
    batch, n_turns, n_triples, _ = kg_enc_input.shape
    B = batch * n_turns * n_triples
    idx_t = kg_enc_input.reshape(batch, n_turns * n_triples, 3)
    idx_t = idx_t.transpose(1, 2, 0)
    head = idx_t[:, 0, :].reshape(B)
    rel = idx_t[:, 1, :].reshape(B)
    tail = idx_t[:, 2, :].reshape(B)
    ent_pad = _pad_table(entity_embedding.T)
    rtab_pad = _pad_table(rel_embedding.T)
    W_pad = jnp.pad(W_mlp.reshape(3 * D, 3, D), ((0, 0), (0, 0), (0, DP - D)))
    W_pad = W_pad.reshape(3 * D, 3 * DP)
    g_et = _make_gather(B, 2)(head, tail, ent_pad)
    g_r = _make_gather(B, 1)(rel, rtab_pad)
    out = _matmul(g_et, g_r, W_pad, b_mlp, n_turns * n_triples, batch)
    out = out.reshape(n_turns * n_triples, 3 * D, batch)
    out = out.transpose(2, 0, 1).reshape(batch, n_turns, n_triples, 3 * D)
    return out


# speedup vs baseline: 6.4141x; 1.4971x over previous
import functools

import jax
import jax.numpy as jnp
from jax import lax
from jax.experimental import pallas as pl
from jax.experimental.pallas import tpu as pltpu
from jax.experimental.pallas import tpu_sc as plsc

NC = 2
NS = 16
NW = NC * NS
CHUNK = 64
D = 300
DP = 384


def _pad_body(xt_ref, o_ref):
    xt = xt_ref[...]
    o_ref[...] = jnp.concatenate(
        [xt.T, jnp.zeros((xt.shape[1], DP - D), jnp.float32)], axis=1)


def _pad_table(tbl_t, rows_per_blk=2048):
    V = tbl_t.shape[1]
    return pl.pallas_call(
        _pad_body,
        grid=(pl.cdiv(V, rows_per_blk),),
        in_specs=[pl.BlockSpec((D, rows_per_blk), lambda i: (0, i))],
        out_specs=pl.BlockSpec((rows_per_blk, DP), lambda i: (i, 0)),
        out_shape=jax.ShapeDtypeStruct((V, DP), jnp.float32),
    )(tbl_t)


def _make_gather(B, n_parts):
    per_w = B // NW
    n_chunks = per_w // CHUNK
    mesh = plsc.VectorSubcoreMesh(core_axis_name="c", subcore_axis_name="s")

    idx_scratch = [pltpu.VMEM((per_w,), jnp.int32)] * n_parts
    buf_scratch = [pltpu.VMEM((CHUNK, DP), jnp.float32)] * n_parts
    sem_scratch = [pltpu.SemaphoreType.DMA] * n_parts

    @functools.partial(
        pl.kernel,
        out_type=jax.ShapeDtypeStruct((n_parts, B, DP), jnp.float32),
        mesh=mesh,
        scratch_types=idx_scratch + buf_scratch + sem_scratch,
    )
    def gather_k(*args):
        idx_hbm = args[:n_parts]
        tbl_hbm = args[n_parts]
        out_hbm = args[n_parts + 1]
        idx_v = args[n_parts + 2:2 * n_parts + 2]
        bufs = args[2 * n_parts + 2:3 * n_parts + 2]
        sems = args[3 * n_parts + 2:]
        wid = lax.axis_index("s") * NC + lax.axis_index("c")
        base = wid * per_w
        for j in range(n_parts):
            pltpu.sync_copy(idx_hbm[j].at[pl.ds(base, per_w)], idx_v[j])

        def body(c, carry):
            off = base + c * CHUNK
            cs = pl.ds(c * CHUNK, CHUNK)
            cps = [pltpu.async_copy(tbl_hbm.at[idx_v[j].at[cs]], bufs[j],
                                    sems[j])
                   for j in range(n_parts)]
            for j in range(n_parts):
                cps[j].wait()
                pltpu.sync_copy(bufs[j], out_hbm.at[j, pl.ds(off, CHUNK)])
            return carry

        lax.fori_loop(0, n_chunks, body, 0)

    return gather_k


def _mm_body(xet_ref, xr_ref, w_ref, b_ref, o_ref):
    w = w_ref[...]
    acc = lax.dot_general(w[:, 0:DP], xet_ref[0],
                          (((1,), (1,)), ((), ())),
                          preferred_element_type=jnp.float32)
    acc += lax.dot_general(w[:, DP:2 * DP], xr_ref[0],
                           (((1,), (1,)), ((), ())),
                           preferred_element_type=jnp.float32)
    acc += lax.dot_general(w[:, 2 * DP:3 * DP], xet_ref[1],
                           (((1,), (1,)), ((), ())),
                           preferred_element_type=jnp.float32)
    acc += b_ref[...]
    o_ref[...] = acc.reshape(1, 3 * D, 8, 128)


def _matmul(g_et, g_r, W_pad, b_mlp, n_triples, batch, interpret=False):
    grid = (n_triples,)
    return pl.pallas_call(
        _mm_body,
        grid=grid,
        in_specs=[
            pl.BlockSpec((2, batch, DP), lambda i: (0, i, 0)),
            pl.BlockSpec((1, batch, DP), lambda i: (0, i, 0)),
            pl.BlockSpec((3 * D, 3 * DP), lambda i: (0, 0)),
            pl.BlockSpec((3 * D, 1), lambda i: (0, 0)),
        ],
        out_specs=pl.BlockSpec((1, 3 * D, 8, 128), lambda i: (i, 0, 0, 0)),
        out_shape=jax.ShapeDtypeStruct(
            (n_triples, 3 * D, batch // 128, 128), jnp.float32),
        interpret=interpret,
    )(g_et, g_r, W_pad, b_mlp.reshape(3 * D, 1))


def kernel(kg_enc_input, entity_embedding, rel_embedding, W_mlp, b_mlp):
    batch, n_turns, n_triples, _ = kg_enc_input.shape
    B = batch * n_turns * n_triples
    idx_t = kg_enc_input.reshape(batch, n_turns * n_triples, 3)
    idx_t = idx_t.transpose(1, 2, 0)
    head = idx_t[:, 0, :].reshape(B)
    rel = idx_t[:, 1, :].reshape(B)
    tail = idx_t[:, 2, :].reshape(B)
    ent_pad = _pad_table(entity_embedding.T)
    rtab_pad = _pad_table(rel_embedding.T)
    W_pad = jnp.pad(W_mlp.reshape(3 * D, 3, D), ((0, 0), (0, 0), (0, DP - D)))
    W_pad = W_pad.reshape(3 * D, 3 * DP)
    g_et = _make_gather(B, 2)(head, tail, ent_pad)
    g_r = _make_gather(B, 1)(rel, rtab_pad)
    out = _matmul(g_et, g_r, W_pad, b_mlp, n_turns * n_triples, batch)
    out = out.transpose(2, 3, 0, 1).reshape(batch, n_turns, n_triples, 3 * D)
    return out


# speedup vs baseline: 6.5002x; 1.0134x over previous
import functools

import jax
import jax.numpy as jnp
from jax import lax
from jax.experimental import pallas as pl
from jax.experimental.pallas import tpu as pltpu
from jax.experimental.pallas import tpu_sc as plsc

NC = 2
NS = 16
NW = NC * NS
CHUNK = 40
D = 300
DP = 384


def _pad_body(xt_ref, o_ref):
    xt = xt_ref[...]
    o_ref[...] = jnp.concatenate(
        [xt.T, jnp.zeros((xt.shape[1], DP - D), jnp.float32)], axis=1)


def _pad_table(tbl_t, rows_per_blk=2048):
    V = tbl_t.shape[1]
    return pl.pallas_call(
        _pad_body,
        grid=(pl.cdiv(V, rows_per_blk),),
        in_specs=[pl.BlockSpec((D, rows_per_blk), lambda i: (0, i))],
        out_specs=pl.BlockSpec((rows_per_blk, DP), lambda i: (i, 0)),
        out_shape=jax.ShapeDtypeStruct((V, DP), jnp.float32),
    )(tbl_t)


def _make_gather(B, n_parts):
    per_w = B // NW
    n_chunks = per_w // CHUNK
    mesh = plsc.VectorSubcoreMesh(core_axis_name="c", subcore_axis_name="s")

    assert n_chunks % 2 == 0
    idx_scratch = [pltpu.VMEM((per_w,), jnp.int32)] * n_parts
    buf_scratch = [pltpu.VMEM((2, CHUNK, DP), jnp.float32)] * n_parts
    sem_scratch = [pltpu.SemaphoreType.DMA((2,))] * n_parts

    @functools.partial(
        pl.kernel,
        out_type=jax.ShapeDtypeStruct((n_parts, B, DP), jnp.float32),
        mesh=mesh,
        scratch_types=idx_scratch + buf_scratch + sem_scratch,
    )
    def gather_k(*args):
        idx_hbm = args[:n_parts]
        tbl_hbm = args[n_parts]
        out_hbm = args[n_parts + 1]
        idx_v = args[n_parts + 2:2 * n_parts + 2]
        bufs = args[2 * n_parts + 2:3 * n_parts + 2]
        sems = args[3 * n_parts + 2:]
        wid = lax.axis_index("s") * NC + lax.axis_index("c")
        base = wid * per_w
        for j in range(n_parts):
            pltpu.sync_copy(idx_hbm[j].at[pl.ds(base, per_w)], idx_v[j])

        def start(c, p):
            cs = pl.ds(c * CHUNK, CHUNK)
            for j in range(n_parts):
                pltpu.async_copy(tbl_hbm.at[idx_v[j].at[cs]], bufs[j].at[p],
                                 sems[j].at[p])

        def finish(c, p):
            off = base + c * CHUNK
            for j in range(n_parts):
                pltpu.make_async_copy(tbl_hbm.at[idx_v[j].at[pl.ds(0, CHUNK)]],
                                      bufs[j].at[p], sems[j].at[p]).wait()
                pltpu.sync_copy(bufs[j].at[p], out_hbm.at[j, pl.ds(off, CHUNK)])

        start(0, 0)

        def body(i, carry):
            c0 = 2 * i
            start(c0 + 1, 1)
            finish(c0, 0)

            @pl.when(i < n_chunks // 2 - 1)
            def _():
                start(c0 + 2, 0)

            finish(c0 + 1, 1)
            return carry

        lax.fori_loop(0, n_chunks // 2, body, 0)

    return gather_k


def _mm_body(xet_ref, xr_ref, w_ref, b_ref, o_ref):
    w = w_ref[...]
    acc = lax.dot_general(w[:, 0:DP], xet_ref[0],
                          (((1,), (1,)), ((), ())),
                          preferred_element_type=jnp.float32)
    acc += lax.dot_general(w[:, DP:2 * DP], xr_ref[0],
                           (((1,), (1,)), ((), ())),
                           preferred_element_type=jnp.float32)
    acc += lax.dot_general(w[:, 2 * DP:3 * DP], xet_ref[1],
                           (((1,), (1,)), ((), ())),
                           preferred_element_type=jnp.float32)
    acc += b_ref[...]
    o_ref[...] = acc.reshape(1, 3 * D, 8, 128)


def _matmul(g_et, g_r, W_pad, b_mlp, n_triples, batch, interpret=False):
    grid = (n_triples,)
    return pl.pallas_call(
        _mm_body,
        grid=grid,
        in_specs=[
            pl.BlockSpec((2, batch, DP), lambda i: (0, i, 0)),
            pl.BlockSpec((1, batch, DP), lambda i: (0, i, 0)),
            pl.BlockSpec((3 * D, 3 * DP), lambda i: (0, 0)),
            pl.BlockSpec((3 * D, 1), lambda i: (0, 0)),
        ],
        out_specs=pl.BlockSpec((1, 3 * D, 8, 128), lambda i: (i, 0, 0, 0)),
        out_shape=jax.ShapeDtypeStruct(
            (n_triples, 3 * D, batch // 128, 128), jnp.float32),
        interpret=interpret,
    )(g_et, g_r, W_pad, b_mlp.reshape(3 * D, 1))


def kernel(kg_enc_input, entity_embedding, rel_embedding, W_mlp, b_mlp):
    batch, n_turns, n_triples, _ = kg_enc_input.shape
    B = batch * n_turns * n_triples
    idx_t = kg_enc_input.reshape(batch, n_turns * n_triples, 3)
    idx_t = idx_t.transpose(1, 2, 0)
    head = idx_t[:, 0, :].reshape(B)
    rel = idx_t[:, 1, :].reshape(B)
    tail = idx_t[:, 2, :].reshape(B)
    ent_pad = _pad_table(entity_embedding.T)
    rtab_pad = _pad_table(rel_embedding.T)
    W_pad = jnp.pad(W_mlp.reshape(3 * D, 3, D), ((0, 0), (0, 0), (0, DP - D)))
    W_pad = W_pad.reshape(3 * D, 3 * DP)
    g_et = _make_gather(B, 2)(head, tail, ent_pad)
    g_r = _make_gather(B, 1)(rel, rtab_pad)
    out = _matmul(g_et, g_r, W_pad, b_mlp, n_turns * n_triples, batch)
    out = out.transpose(2, 3, 0, 1).reshape(batch, n_turns, n_triples, 3 * D)
    return out


# speedup vs baseline: 7.7709x; 1.1955x over previous
import functools

import jax
import jax.numpy as jnp
from jax import lax
from jax.experimental import pallas as pl
from jax.experimental.pallas import tpu as pltpu
from jax.experimental.pallas import tpu_sc as plsc

NC = 2
NS = 16
NW = NC * NS
CHUNK = 40
D = 300
DB = 512
DP = 256


def _pack(padded):
    lo = lax.bitcast_convert_type(padded[:, :DP], jnp.uint16)
    hi = lax.bitcast_convert_type(padded[:, DP:], jnp.uint16)
    word = lo.astype(jnp.uint32) | (hi.astype(jnp.uint32) << 16)
    return lax.bitcast_convert_type(word, jnp.float32)


def _pad_body(xt_ref, o_ref):
    xt = xt_ref[...].astype(jnp.bfloat16)
    rows = xt.shape[1]
    padded = jnp.concatenate(
        [xt.T, jnp.zeros((rows, DB - D), jnp.bfloat16)], axis=1)
    o_ref[...] = _pack(padded)


def _pad_table(tbl_t, rows_per_blk=2048):
    V = tbl_t.shape[1]
    return pl.pallas_call(
        _pad_body,
        grid=(pl.cdiv(V, rows_per_blk),),
        in_specs=[pl.BlockSpec((D, rows_per_blk), lambda i: (0, i))],
        out_specs=pl.BlockSpec((rows_per_blk, DP), lambda i: (i, 0)),
        out_shape=jax.ShapeDtypeStruct((V, DP), jnp.float32),
    )(tbl_t)


def _make_gather(B, n_parts):
    per_w = B // NW
    n_chunks = per_w // CHUNK
    mesh = plsc.VectorSubcoreMesh(core_axis_name="c", subcore_axis_name="s")

    assert n_chunks % 2 == 0
    idx_scratch = [pltpu.VMEM((per_w,), jnp.int32)] * n_parts
    buf_scratch = [pltpu.VMEM((2, CHUNK, DP), jnp.float32)] * n_parts
    sem_scratch = [pltpu.SemaphoreType.DMA((2,))] * n_parts

    @functools.partial(
        pl.kernel,
        out_type=jax.ShapeDtypeStruct((n_parts, B, DP), jnp.float32),
        mesh=mesh,
        scratch_types=idx_scratch + buf_scratch + sem_scratch,
    )
    def gather_k(*args):
        idx_hbm = args[:n_parts]
        tbl_hbm = args[n_parts]
        out_hbm = args[n_parts + 1]
        idx_v = args[n_parts + 2:2 * n_parts + 2]
        bufs = args[2 * n_parts + 2:3 * n_parts + 2]
        sems = args[3 * n_parts + 2:]
        wid = lax.axis_index("s") * NC + lax.axis_index("c")
        base = wid * per_w
        for j in range(n_parts):
            pltpu.sync_copy(idx_hbm[j].at[pl.ds(base, per_w)], idx_v[j])

        def start(c, p):
            cs = pl.ds(c * CHUNK, CHUNK)
            for j in range(n_parts):
                pltpu.async_copy(tbl_hbm.at[idx_v[j].at[cs]], bufs[j].at[p],
                                 sems[j].at[p])

        def finish(c, p):
            off = base + c * CHUNK
            for j in range(n_parts):
                pltpu.make_async_copy(tbl_hbm.at[idx_v[j].at[pl.ds(0, CHUNK)]],
                                      bufs[j].at[p], sems[j].at[p]).wait()
                pltpu.sync_copy(bufs[j].at[p], out_hbm.at[j, pl.ds(off, CHUNK)])

        start(0, 0)

        def body(i, carry):
            c0 = 2 * i
            start(c0 + 1, 1)
            finish(c0, 0)

            @pl.when(i < n_chunks // 2 - 1)
            def _():
                start(c0 + 2, 0)

            finish(c0 + 1, 1)
            return carry

        lax.fori_loop(0, n_chunks // 2, body, 0)

    return gather_k


def _unpack(x):
    word = lax.bitcast_convert_type(x, jnp.uint32)
    lo = lax.bitcast_convert_type((word & 0xFFFF).astype(jnp.uint16),
                                  jnp.bfloat16)
    hi = lax.bitcast_convert_type((word >> 16).astype(jnp.uint16),
                                  jnp.bfloat16)
    return jnp.concatenate([lo, hi], axis=1)


def _mm_body(xet_ref, xr_ref, w_ref, b_ref, o_ref):
    w = w_ref[...]
    acc = lax.dot_general(w[:, 0:DB], _unpack(xet_ref[0]),
                          (((1,), (1,)), ((), ())),
                          preferred_element_type=jnp.float32)
    acc += lax.dot_general(w[:, DB:2 * DB], _unpack(xr_ref[0]),
                           (((1,), (1,)), ((), ())),
                           preferred_element_type=jnp.float32)
    acc += lax.dot_general(w[:, 2 * DB:3 * DB], _unpack(xet_ref[1]),
                           (((1,), (1,)), ((), ())),
                           preferred_element_type=jnp.float32)
    acc += b_ref[...]
    o_ref[...] = acc.reshape(1, 3 * D, 8, 128)


def _matmul(g_et, g_r, W_pad, b_mlp, n_triples, batch, interpret=False):
    grid = (n_triples,)
    return pl.pallas_call(
        _mm_body,
        grid=grid,
        in_specs=[
            pl.BlockSpec((2, batch, DP), lambda i: (0, i, 0)),
            pl.BlockSpec((1, batch, DP), lambda i: (0, i, 0)),
            pl.BlockSpec((3 * D, 3 * DB), lambda i: (0, 0)),
            pl.BlockSpec((3 * D, 1), lambda i: (0, 0)),
        ],
        out_specs=pl.BlockSpec((1, 3 * D, 8, 128), lambda i: (i, 0, 0, 0)),
        out_shape=jax.ShapeDtypeStruct(
            (n_triples, 3 * D, batch // 128, 128), jnp.float32),
        interpret=interpret,
    )(g_et, g_r, W_pad, b_mlp.reshape(3 * D, 1))


def kernel(kg_enc_input, entity_embedding, rel_embedding, W_mlp, b_mlp):
    batch, n_turns, n_triples, _ = kg_enc_input.shape
    B = batch * n_turns * n_triples
    idx_t = kg_enc_input.reshape(batch, n_turns * n_triples, 3)
    idx_t = idx_t.transpose(1, 2, 0)
    head = idx_t[:, 0, :].reshape(B)
    rel = idx_t[:, 1, :].reshape(B)
    tail = idx_t[:, 2, :].reshape(B)
    ent_pad = _pad_table(entity_embedding.T)
    rtab_pad = _pad_table(rel_embedding.T)
    W_pad = jnp.pad(W_mlp.reshape(3 * D, 3, D), ((0, 0), (0, 0), (0, DB - D)))
    W_pad = W_pad.reshape(3 * D, 3 * DB).astype(jnp.bfloat16)
    g_et = _make_gather(B, 2)(head, tail, ent_pad)
    g_r = _make_gather(B, 1)(rel, rtab_pad)
    out = _matmul(g_et, g_r, W_pad, b_mlp, n_turns * n_triples, batch)
    out = out.transpose(2, 3, 0, 1).reshape(batch, n_turns, n_triples, 3 * D)
    return out


# speedup vs baseline: 7.8936x; 1.0158x over previous
import functools

import jax
import jax.numpy as jnp
from jax import lax
from jax.experimental import pallas as pl
from jax.experimental.pallas import tpu as pltpu
from jax.experimental.pallas import tpu_sc as plsc

NC = 2
NS = 16
NW = NC * NS
CHUNK = 40
D = 300
DB = 512
DP = 256


def _pack(padded):
    lo = lax.bitcast_convert_type(padded[:, :DP], jnp.uint16)
    hi = lax.bitcast_convert_type(padded[:, DP:], jnp.uint16)
    word = lo.astype(jnp.uint32) | (hi.astype(jnp.uint32) << 16)
    return lax.bitcast_convert_type(word, jnp.float32)


def _pad_body(xt_ref, o_ref):
    xt = xt_ref[...].astype(jnp.bfloat16)
    rows = xt.shape[1]
    padded = jnp.concatenate(
        [xt.T, jnp.zeros((rows, DB - D), jnp.bfloat16)], axis=1)
    o_ref[...] = _pack(padded)


def _pad_table(tbl_t, rows_per_blk=2048):
    V = tbl_t.shape[1]
    return pl.pallas_call(
        _pad_body,
        grid=(pl.cdiv(V, rows_per_blk),),
        in_specs=[pl.BlockSpec((D, rows_per_blk), lambda i: (0, i))],
        out_specs=pl.BlockSpec((rows_per_blk, DP), lambda i: (i, 0)),
        out_shape=jax.ShapeDtypeStruct((V, DP), jnp.float32),
    )(tbl_t)


def _make_gather(B, n_parts):
    per_w = B // NW
    n_chunks = per_w // CHUNK
    mesh = plsc.VectorSubcoreMesh(core_axis_name="c", subcore_axis_name="s")

    assert n_chunks % 2 == 0
    idx_scratch = [pltpu.VMEM((per_w,), jnp.int32)] * n_parts
    buf_scratch = [pltpu.VMEM((2, CHUNK, DP), jnp.float32)] * n_parts
    sem_scratch = [pltpu.SemaphoreType.DMA((2,))] * n_parts

    @functools.partial(
        pl.kernel,
        out_type=jax.ShapeDtypeStruct((n_parts, B, DP), jnp.float32),
        mesh=mesh,
        scratch_types=idx_scratch + buf_scratch + sem_scratch,
    )
    def gather_k(*args):
        idx_hbm = args[:n_parts]
        tbl_hbm = args[n_parts]
        out_hbm = args[n_parts + 1]
        idx_v = args[n_parts + 2:2 * n_parts + 2]
        bufs = args[2 * n_parts + 2:3 * n_parts + 2]
        sems = args[3 * n_parts + 2:]
        wid = lax.axis_index("s") * NC + lax.axis_index("c")
        base = wid * per_w
        for j in range(n_parts):
            pltpu.sync_copy(idx_hbm[j].at[pl.ds(base, per_w)], idx_v[j])

        def start(c, p):
            cs = pl.ds(c * CHUNK, CHUNK)
            for j in range(n_parts):
                pltpu.async_copy(tbl_hbm.at[idx_v[j].at[cs]], bufs[j].at[p],
                                 sems[j].at[p])

        def finish(c, p):
            off = base + c * CHUNK
            for j in range(n_parts):
                pltpu.make_async_copy(tbl_hbm.at[idx_v[j].at[pl.ds(0, CHUNK)]],
                                      bufs[j].at[p], sems[j].at[p]).wait()
                pltpu.sync_copy(bufs[j].at[p], out_hbm.at[j, pl.ds(off, CHUNK)])

        start(0, 0)

        def body(i, carry):
            c0 = 2 * i
            start(c0 + 1, 1)
            finish(c0, 0)

            @pl.when(i < n_chunks // 2 - 1)
            def _():
                start(c0 + 2, 0)

            finish(c0 + 1, 1)
            return carry

        lax.fori_loop(0, n_chunks // 2, body, 0)

    return gather_k


def _unpack(x):
    word = lax.bitcast_convert_type(x, jnp.uint32)
    lo = lax.bitcast_convert_type(word.astype(jnp.uint16), jnp.bfloat16)
    hi = lax.bitcast_convert_type((word >> 16).astype(jnp.uint16),
                                  jnp.bfloat16)
    return lo, hi


def _mm_body(xet_ref, xr_ref, w_ref, b_ref, o_ref):
    w = w_ref[...]
    dn = (((1,), (1,)), ((), ()))
    acc = None
    for j, x in ((0, xet_ref[0]), (1, xr_ref[0]), (2, xet_ref[1])):
        lo, hi = _unpack(x)
        d = lax.dot_general(w[:, j * DB:j * DB + DP], lo, dn,
                            preferred_element_type=jnp.float32)
        d += lax.dot_general(w[:, j * DB + DP:(j + 1) * DB], hi, dn,
                             preferred_element_type=jnp.float32)
        acc = d if acc is None else acc + d
    acc += b_ref[...]
    o_ref[...] = acc.reshape(1, 3 * D, 8, 128)


def _matmul(g_et, g_r, W_pad, b_mlp, n_triples, batch, interpret=False):
    grid = (n_triples,)
    return pl.pallas_call(
        _mm_body,
        grid=grid,
        in_specs=[
            pl.BlockSpec((2, batch, DP), lambda i: (0, i, 0)),
            pl.BlockSpec((1, batch, DP), lambda i: (0, i, 0)),
            pl.BlockSpec((3 * D, 3 * DB), lambda i: (0, 0)),
            pl.BlockSpec((3 * D, 1), lambda i: (0, 0)),
        ],
        out_specs=pl.BlockSpec((1, 3 * D, 8, 128), lambda i: (i, 0, 0, 0)),
        out_shape=jax.ShapeDtypeStruct(
            (n_triples, 3 * D, batch // 128, 128), jnp.float32),
        interpret=interpret,
    )(g_et, g_r, W_pad, b_mlp.reshape(3 * D, 1))


def kernel(kg_enc_input, entity_embedding, rel_embedding, W_mlp, b_mlp):
    batch, n_turns, n_triples, _ = kg_enc_input.shape
    B = batch * n_turns * n_triples
    idx_t = kg_enc_input.reshape(batch, n_turns * n_triples, 3)
    idx_t = idx_t.transpose(1, 2, 0)
    head = idx_t[:, 0, :].reshape(B)
    rel = idx_t[:, 1, :].reshape(B)
    tail = idx_t[:, 2, :].reshape(B)
    ent_pad = _pad_table(entity_embedding.T)
    rtab_pad = _pad_table(rel_embedding.T)
    W_pad = jnp.pad(W_mlp.reshape(3 * D, 3, D), ((0, 0), (0, 0), (0, DB - D)))
    W_pad = W_pad.reshape(3 * D, 3 * DB).astype(jnp.bfloat16)
    g_et = _make_gather(B, 2)(head, tail, ent_pad)
    g_r = _make_gather(B, 1)(rel, rtab_pad)
    out = _matmul(g_et, g_r, W_pad, b_mlp, n_turns * n_triples, batch)
    out = out.transpose(2, 3, 0, 1).reshape(batch, n_turns, n_triples, 3 * D)
    return out
